# baseline (speedup 1.0000x reference)
import jax
import jax.numpy as jnp
import numpy as np
from jax import lax
from jax.experimental import pallas as pl
from jax.experimental.pallas import tpu as pltpu
from jax.experimental.pallas import tpu_sc as plsc

EMB_DIM = 128
BATCH_N = 16384
N_CORES = 2
N_SUBCORES = 16
LANES = 16
SEGS = EMB_DIM // LANES
N_WORKERS = N_CORES * N_SUBCORES
ROWS_PER_WORKER = BATCH_N // N_WORKERS
CHUNK = 64
N_CHUNKS = ROWS_PER_WORKER // CHUNK
GROUPS = CHUNK // LANES
PACKED_W = EMB_DIM // 2

_PERM = np.array([32 * b + 16 * p + i
                  for b in range(EMB_DIM // 32)
                  for i in range(16)
                  for p in range(2)], dtype=np.int32)


def _pack_bf16_table(w):
    wp = w[:, _PERM].astype(jnp.bfloat16)
    return lax.bitcast_convert_type(
        wp.reshape(w.shape[0], PACKED_W, 2), jnp.int32)


def _sc_body(h_hbm, t_hbm, r_hbm, ent_hbm, reln_hbm, out_hbm,
             hi_all, ti_all, ri_all,
             hr0, tr0, rn0, hr1, tr1, rn1, score_all, sem0, sem1):
    wid = lax.axis_index("s") * N_CORES + lax.axis_index("c")
    base = wid * ROWS_PER_WORKER
    lanes = lax.iota(jnp.int32, LANES)

    pltpu.sync_copy(h_hbm.at[pl.ds(base, ROWS_PER_WORKER)], hi_all)
    pltpu.sync_copy(t_hbm.at[pl.ds(base, ROWS_PER_WORKER)], ti_all)
    pltpu.sync_copy(r_hbm.at[pl.ds(base, ROWS_PER_WORKER)], ri_all)

    def fire(ck, hr, tr, rnr, sem):
        sl = pl.ds(ck * CHUNK, CHUNK)
        pltpu.async_copy(ent_hbm.at[hi_all.at[sl]], hr, sem)
        pltpu.async_copy(ent_hbm.at[ti_all.at[sl]], tr, sem)
        pltpu.async_copy(reln_hbm.at[ri_all.at[sl]], rnr, sem)

    def wait3(hr, tr, rnr, sem):
        sl = pl.ds(0, CHUNK)
        pltpu.make_async_copy(ent_hbm.at[hi_all.at[sl]], hr, sem).wait()
        pltpu.make_async_copy(ent_hbm.at[ti_all.at[sl]], tr, sem).wait()
        pltpu.make_async_copy(reln_hbm.at[ri_all.at[sl]], rnr, sem).wait()

    def compute(ck, hrows, trows, rnrows):
        def group_body(g, carry2):
            score_vec = jnp.zeros((LANES,), jnp.float32)
            for k in range(LANES):
                i = g * LANES + k
                d = [hrows[i, pl.ds(j * LANES, LANES)]
                     - trows[i, pl.ds(j * LANES, LANES)]
                     for j in range(SEGS)]
                n = []
                rv = []
                for blk in range(SEGS // 2):
                    rw = plsc.bitcast(rnrows[i, pl.ds(blk * LANES, LANES)],
                                      jnp.bfloat16)
                    ra, rb = plsc.unpack(rw, format=plsc.PackFormat.INTERLEAVED)
                    rv.extend((ra, rb))
                    nw = plsc.bitcast(
                        rnrows[i, pl.ds(PACKED_W + blk * LANES, LANES)],
                        jnp.bfloat16)
                    na, nb = plsc.unpack(nw, format=plsc.PackFormat.INTERLEAVED)
                    n.extend((na, nb))
                dot = d[0] * n[0]
                for j in range(1, SEGS):
                    dot = dot + d[j] * n[j]
                s = jnp.sum(dot)
                acc = jnp.zeros((LANES,), jnp.float32)
                for j in range(SEGS):
                    acc = acc + jnp.abs(d[j] + rv[j] - s * n[j])
                score_vec = jnp.where(lanes == k, jnp.sum(acc), score_vec)
            score_all[pl.ds(ck * CHUNK + g * LANES, LANES)] = score_vec
            return carry2

        lax.fori_loop(0, GROUPS, group_body, 0)

    fire(0, hr0, tr0, rn0, sem0)

    def pair_body(p, carry):
        c0 = 2 * p
        fire(c0 + 1, hr1, tr1, rn1, sem1)
        wait3(hr0, tr0, rn0, sem0)
        compute(c0, hr0, tr0, rn0)

        @pl.when(p < N_CHUNKS // 2 - 1)
        def _():
            fire(c0 + 2, hr0, tr0, rn0, sem0)

        wait3(hr1, tr1, rn1, sem1)
        compute(c0 + 1, hr1, tr1, rn1)
        return carry

    lax.fori_loop(0, N_CHUNKS // 2, pair_body, 0)
    pltpu.sync_copy(score_all, out_hbm.at[pl.ds(base, ROWS_PER_WORKER)])


def kernel(h, t, r, ent_weight, rel_weight, norm_weight):
    mesh = plsc.VectorSubcoreMesh(core_axis_name="c", subcore_axis_name="s")
    run = pl.kernel(
        _sc_body,
        out_type=jax.ShapeDtypeStruct((BATCH_N,), jnp.float32),
        mesh=mesh,
        compiler_params=pltpu.CompilerParams(needs_layout_passes=False),
        scratch_types=[
            pltpu.VMEM((ROWS_PER_WORKER,), jnp.int32),
            pltpu.VMEM((ROWS_PER_WORKER,), jnp.int32),
            pltpu.VMEM((ROWS_PER_WORKER,), jnp.int32),
            pltpu.VMEM((CHUNK, EMB_DIM), jnp.float32),
            pltpu.VMEM((CHUNK, EMB_DIM), jnp.float32),
            pltpu.VMEM((CHUNK, 2 * PACKED_W), jnp.int32),
            pltpu.VMEM((CHUNK, EMB_DIM), jnp.float32),
            pltpu.VMEM((CHUNK, EMB_DIM), jnp.float32),
            pltpu.VMEM((CHUNK, 2 * PACKED_W), jnp.int32),
            pltpu.VMEM((ROWS_PER_WORKER,), jnp.float32),
            pltpu.SemaphoreType.DMA,
            pltpu.SemaphoreType.DMA,
        ],
    )
    reln = jnp.concatenate(
        [_pack_bf16_table(rel_weight), _pack_bf16_table(norm_weight)], axis=1)
    return run(h.astype(jnp.int32), t.astype(jnp.int32), r.astype(jnp.int32),
               ent_weight, reln)
